# baseline (speedup 1.0000x reference)
import jax
import jax.numpy as jnp
from jax import lax
from jax.experimental import pallas as pl
from jax.experimental.pallas import tpu as pltpu

_D = 128
_TR = 80
_NT = 50
_CH = _TR * _NT
_NBUF = 3


def _stream_kernel(x1_hbm, x2_hbm, w_ref, b_ref, y1_hbm, y2_hbm,
                   xbufs, ybufs, in_sems, out_sems):
    nch = x1_hbm.shape[0]
    tasks = []
    for k in range(nch):
        tasks.append((x1_hbm, y1_hbm, 0, k))
        tasks.append((x2_hbm, y2_hbm, 1, k))
    nt = len(tasks)

    def read(t, slot):
        src, _, _, c = tasks[t]
        return pltpu.make_async_copy(src.at[c], xbufs.at[slot], in_sems.at[slot])

    def write(t, slot):
        _, dst, _, c = tasks[t]
        return pltpu.make_async_copy(ybufs.at[slot], dst.at[c], out_sems.at[slot])

    for t in range(_NBUF):
        read(t, t).start()
    for t in range(nt):
        slot = t % _NBUF
        read(t, slot).wait()
        if t >= _NBUF:
            write(t - _NBUF, slot).wait()
        inp = tasks[t][2]
        wv = w_ref[inp][None, :]
        bv = b_ref[inp][None, :]

        def tile_body(r, carry):
            ybufs[slot, r] = jnp.maximum(xbufs[slot, r], 0.0) * wv + bv
            return carry

        lax.fori_loop(0, _NT, tile_body, 0)
        write(t, slot).start()
        if t + _NBUF < nt:
            read(t + _NBUF, slot).start()
    for t in range(nt - _NBUF, nt):
        write(t, t % _NBUF).wait()


def kernel(x1, x2, W, b):
    n, d = x1.shape
    nch = n // _CH
    wstack = W.reshape(2, d)
    bstack = b.reshape(2, d)
    x1v = x1.reshape(nch, _NT, _TR, d)
    x2v = x2.reshape(nch, _NT, _TR, d)
    y1, y2 = pl.pallas_call(
        _stream_kernel,
        in_specs=[
            pl.BlockSpec(memory_space=pltpu.MemorySpace.HBM),
            pl.BlockSpec(memory_space=pltpu.MemorySpace.HBM),
            pl.BlockSpec(memory_space=pltpu.MemorySpace.VMEM),
            pl.BlockSpec(memory_space=pltpu.MemorySpace.VMEM),
        ],
        out_specs=[
            pl.BlockSpec(memory_space=pltpu.MemorySpace.HBM),
            pl.BlockSpec(memory_space=pltpu.MemorySpace.HBM),
        ],
        out_shape=[jax.ShapeDtypeStruct((nch, _NT, _TR, d), x1.dtype)] * 2,
        scratch_shapes=[
            pltpu.VMEM((_NBUF, _NT, _TR, d), jnp.float32),
            pltpu.VMEM((_NBUF, _NT, _TR, d), jnp.float32),
            pltpu.SemaphoreType.DMA((_NBUF,)),
            pltpu.SemaphoreType.DMA((_NBUF,)),
        ],
    )(x1v, x2v, wstack, bstack)
    return (y1.reshape(n, d), y2.reshape(n, d))


# speedup vs baseline: 1.0184x; 1.0184x over previous
import jax
import jax.numpy as jnp
from jax.experimental import pallas as pl


def _ew_kernel(x_ref, w_ref, b_ref, y_ref):
    y_ref[...] = jnp.maximum(x_ref[...], 0.0) * w_ref[...] + b_ref[...]


def _stream(x, w, b, block_rows):
    n, d = x.shape
    bs_x = pl.BlockSpec((block_rows, d), lambda i: (i, 0))
    bs_w = pl.BlockSpec((1, d), lambda i: (0, 0))
    return pl.pallas_call(
        _ew_kernel,
        grid=(n // block_rows,),
        in_specs=[bs_x, bs_w, bs_w],
        out_specs=bs_x,
        out_shape=jax.ShapeDtypeStruct((n, d), x.dtype),
    )(x, w.reshape(1, d), b.reshape(1, d))


def kernel(x1, x2, W, b):
    n, d = x1.shape
    y1 = _stream(x1, W[:d], b[:d], 20000)
    y2 = _stream(x2, W[d:], b[d:], 20000)
    return (y1, y2)


# speedup vs baseline: 1.0249x; 1.0064x over previous
import jax
import jax.numpy as jnp
from jax.experimental import pallas as pl


def _ew_kernel(x_ref, w_ref, b_ref, y_ref):
    y_ref[...] = jnp.maximum(x_ref[...], 0.0) * w_ref[...] + b_ref[...]


def _stream(x, w, b, block_rows):
    n, d = x.shape
    bs_x = pl.BlockSpec((block_rows, d), lambda i: (i, 0))
    bs_w = pl.BlockSpec((1, d), lambda i: (0, 0))
    return pl.pallas_call(
        _ew_kernel,
        grid=(n // block_rows,),
        in_specs=[bs_x, bs_w, bs_w],
        out_specs=bs_x,
        out_shape=jax.ShapeDtypeStruct((n, d), x.dtype),
    )(x, w.reshape(1, d), b.reshape(1, d))


def kernel(x1, x2, W, b):
    n, d = x1.shape
    y1 = _stream(x1, W[:d], b[:d], 25000)
    y2 = _stream(x2, W[d:], b[d:], 25000)
    return (y1, y2)
